# baseline (speedup 1.0000x reference)
import jax
import jax.numpy as jnp
import numpy as np
from jax.experimental import pallas as pl

_EPSILON = float(np.finfo(np.float32).tiny)
_B, _N, _C, _K = 8, 2048, 64, 1024


def _ln(x, g, b, eps=1e-5):
    m = jnp.mean(x, axis=-1, keepdims=True)
    v = jnp.var(x, axis=-1, keepdims=True)
    return (x - m) / jnp.sqrt(v + eps) * g + b


def _body(xyzs_ref, feats_ref, w_ref, W1_ref, b1_ref, g1_ref, be1_ref,
          W2_ref, b2_ref, xyzs_o_ref, feats_o_ref, idx_o_ref):
    W1 = W1_ref[...]
    W2 = W2_ref[...]

    hs = []
    for b in range(_B):
        fb = feats_ref[b]
        z1 = jax.lax.dot_general(fb, W1, (((0,), (0,)), ((), ()))) + b1_ref[...]
        h1 = jax.nn.relu(_ln(z1, g1_ref[...], be1_ref[...]))
        hs.append(jnp.dot(h1, W2) + b2_ref[...])

    w0 = w_ref[...]
    lane = jax.lax.broadcasted_iota(jnp.int32, (_B, _N), 1)

    def step(k, y):
        x_max = jnp.max(y, axis=-1, keepdims=True)
        u = jnp.exp(y - x_max)
        s = jnp.sum(u, axis=-1, keepdims=True)
        sm = u / s
        smax = 1.0 / s
        p = jnp.min(jnp.where(sm == smax, lane, _N), axis=-1, keepdims=True)
        return jnp.where(lane == p, -10000.0 - k.astype(jnp.float32), y)

    y_fin = jax.lax.fori_loop(0, _K, step, w0 / 0.1)
    steps = -10000.0 - y_fin

    _KC = 1024
    iota_row = jax.lax.broadcasted_iota(jnp.int32, (1, _N), 1).astype(jnp.float32)
    kio = jax.lax.broadcasted_iota(jnp.int32, (_KC, _N), 0).astype(jnp.float32)
    hi = jax.lax.Precision.HIGHEST
    for b in range(_B):
        srow = steps[b:b + 1, :]
        for kc in range(0, _K, _KC):
            onehot = (jnp.broadcast_to(srow, (_KC, _N))
                      == kio + jnp.float32(kc)).astype(jnp.float32)
            feats_o_ref[b, :, kc:kc + _KC] = jax.lax.dot_general(
                hs[b], onehot, (((0,), (1,)), ((), ())), precision=hi)
            xyzs_o_ref[b, :, kc:kc + _KC] = jax.lax.dot_general(
                xyzs_ref[b], onehot, (((1,), (1,)), ((), ())), precision=hi)
            idxf = jax.lax.dot_general(
                iota_row, onehot, (((1,), (1,)), ((), ())), precision=hi)
            idx_o_ref[b:b + 1, kc:kc + _KC] = idxf.astype(jnp.int32)


def kernel(xyzs, features, W1, b1, g1, be1, W2, b2, Wd1, bd1, gd1, bed1, Wd2, bd2):
    f = jnp.transpose(features, (0, 2, 1))
    h = jax.nn.relu(_ln(f @ W1 + b1, g1, be1))
    h = h @ W2 + b2
    wsc = jax.nn.relu(_ln(h @ Wd1 + bd1, gd1, bed1))
    wsc = (wsc @ Wd2 + bd2)[:, :, 0]

    row = lambda v: v.reshape(1, -1)
    xyzs_t, feats_o, idx = pl.pallas_call(
        _body,
        out_shape=(
            jax.ShapeDtypeStruct((_B, 3, _K), jnp.float32),
            jax.ShapeDtypeStruct((_B, _C, _K), jnp.float32),
            jax.ShapeDtypeStruct((_B, _K), jnp.int32),
        ),
    )(jnp.transpose(xyzs, (0, 2, 1)), features, wsc,
      W1, row(b1), row(g1), row(be1), W2, row(b2))
    return jnp.transpose(xyzs_t, (0, 2, 1)), feats_o, idx


# speedup vs baseline: 1.2290x; 1.2290x over previous
import jax
import jax.numpy as jnp
import numpy as np
from jax.experimental import pallas as pl

_EPSILON = float(np.finfo(np.float32).tiny)
_B, _N, _C, _K = 8, 2048, 64, 1024


def _ln(x, g, b, eps=1e-5):
    m = jnp.mean(x, axis=-1, keepdims=True)
    v = jnp.var(x, axis=-1, keepdims=True)
    return (x - m) / jnp.sqrt(v + eps) * g + b


def _body(xyzs_ref, feats_ref, w_ref, W1_ref, b1_ref, g1_ref, be1_ref,
          W2_ref, b2_ref, xyzs_o_ref, feats_o_ref, idx_o_ref):
    W1 = W1_ref[...]
    W2 = W2_ref[...]

    hs = []
    for b in range(_B):
        fb = feats_ref[b]
        z1 = jax.lax.dot_general(fb, W1, (((0,), (0,)), ((), ()))) + b1_ref[...]
        h1 = jax.nn.relu(_ln(z1, g1_ref[...], be1_ref[...]))
        hs.append(jnp.dot(h1, W2) + b2_ref[...])

    w0 = w_ref[...]
    lane = jax.lax.broadcasted_iota(jnp.int32, (_B, _N), 1)

    kiota = jax.lax.broadcasted_iota(jnp.int32, (_B, _K), 1)

    def step(k, carry):
        y, p_prev, idxacc = carry
        y = jnp.where(lane == p_prev, -8.0e4, y)
        x_max = jnp.max(y, axis=-1, keepdims=True)
        u = jnp.exp(y - x_max)
        s = jnp.sum(u, axis=-1, keepdims=True)
        sm = u / s
        smax = 1.0 / s
        p = jnp.min(jnp.where(sm == smax, lane, _N), axis=-1, keepdims=True)
        idxacc = jnp.where(kiota == k, jnp.broadcast_to(p, (_B, _K)), idxacc)
        return (y, p, idxacc)

    init = (w0 / 0.1, jnp.full((_B, 1), -1, jnp.int32),
            jnp.zeros((_B, _K), jnp.int32))
    _, _, idx = jax.lax.fori_loop(0, _K, step, init)

    hi = jax.lax.Precision.HIGHEST
    for b in range(_B):
        idxrow = idx[b:b + 1, :]
        onehot = (jax.lax.broadcasted_iota(jnp.int32, (_N, _K), 0)
                  == jnp.broadcast_to(idxrow, (_N, _K))).astype(jnp.float32)
        feats_o_ref[b] = jax.lax.dot_general(
            hs[b], onehot, (((0,), (0,)), ((), ())), precision=hi)
        xyzs_o_ref[b] = jax.lax.dot_general(
            xyzs_ref[b], onehot, (((1,), (0,)), ((), ())), precision=hi)
    idx_o_ref[...] = idx


def kernel(xyzs, features, W1, b1, g1, be1, W2, b2, Wd1, bd1, gd1, bed1, Wd2, bd2):
    f = jnp.transpose(features, (0, 2, 1))
    h = jax.nn.relu(_ln(f @ W1 + b1, g1, be1))
    h = h @ W2 + b2
    wsc = jax.nn.relu(_ln(h @ Wd1 + bd1, gd1, bed1))
    wsc = (wsc @ Wd2 + bd2)[:, :, 0]

    row = lambda v: v.reshape(1, -1)
    xyzs_t, feats_o, idx = pl.pallas_call(
        _body,
        out_shape=(
            jax.ShapeDtypeStruct((_B, 3, _K), jnp.float32),
            jax.ShapeDtypeStruct((_B, _C, _K), jnp.float32),
            jax.ShapeDtypeStruct((_B, _K), jnp.int32),
        ),
    )(jnp.transpose(xyzs, (0, 2, 1)), features, wsc,
      W1, row(b1), row(g1), row(be1), W2, row(b2))
    return jnp.transpose(xyzs_t, (0, 2, 1)), feats_o, idx
